# initial kernel scaffold (unmeasured)
import jax
import jax.numpy as jnp
from jax import lax
from jax.experimental import pallas as pl
from jax.experimental.pallas import tpu as pltpu


def kernel(
    x,
):
    def body(*refs):
        pass

    out_shape = jax.ShapeDtypeStruct(..., jnp.float32)
    return pl.pallas_call(body, out_shape=out_shape)(...)


# baseline (device time: 13571 ns/iter reference)
import jax
import jax.numpy as jnp
from jax import lax
from jax.experimental import pallas as pl
from jax.experimental.pallas import tpu as pltpu

N_DEV = 16


def kernel(x):
    m, n = x.shape

    def body(x_ref, out_ref, send_ref, recv_ref, send_sem, recv_sem):
        my = lax.axis_index("i")
        left = jnp.maximum(my - 1, 0)
        right = jnp.minimum(my + 1, N_DEV - 1)

        acc = x_ref[:, :]
        shift = 1
        while shift < m:
            pad = jnp.ones((shift, n), jnp.float32)
            acc = acc * jnp.concatenate([pad, acc[: m - shift, :]], axis=0)
            shift *= 2
        total = acc[m - 1 : m, :]

        @pl.when(my == 0)
        def _():
            recv_ref[:, :] = jnp.ones((1, n), jnp.float32)

        @pl.when(my > 0)
        def _():
            recv_desc = pltpu.make_async_remote_copy(
                src_ref=recv_ref,
                dst_ref=recv_ref,
                send_sem=send_sem,
                recv_sem=recv_sem,
                device_id=(left,),
                device_id_type=pl.DeviceIdType.MESH,
            )
            recv_desc.wait_recv()

        prefix = recv_ref[:, :]

        @pl.when(my < N_DEV - 1)
        def _():
            send_ref[:, :] = prefix * total
            send_desc = pltpu.make_async_remote_copy(
                src_ref=send_ref,
                dst_ref=recv_ref,
                send_sem=send_sem,
                recv_sem=recv_sem,
                device_id=(right,),
                device_id_type=pl.DeviceIdType.MESH,
            )
            send_desc.start()
            send_desc.wait_send()

        out_ref[:, :] = acc * prefix

    return pl.pallas_call(
        body,
        out_shape=jax.ShapeDtypeStruct((m, n), jnp.float32),
        in_specs=[pl.BlockSpec(memory_space=pltpu.VMEM)],
        out_specs=pl.BlockSpec(memory_space=pltpu.VMEM),
        scratch_shapes=[
            pltpu.VMEM((1, n), jnp.float32),
            pltpu.VMEM((1, n), jnp.float32),
            pltpu.SemaphoreType.DMA,
            pltpu.SemaphoreType.DMA,
        ],
    )(x)


# device time: 10949 ns/iter; 1.2395x vs baseline; 1.2395x over previous
import jax
import jax.numpy as jnp
from jax import lax
from jax.experimental import pallas as pl
from jax.experimental.pallas import tpu as pltpu

N_DEV = 16


def kernel(x):
    m, n = x.shape

    def body(x_ref, out_ref, totals_ref, send_sems, recv_sems):
        my = lax.axis_index("i")

        vals = x_ref[:, :]
        tot = vals
        rows = m
        while rows > 1:
            half = rows // 2
            tot = tot[:half, :] * tot[half:rows, :]
            rows = half
        total = tot[0:1, :]
        totals_ref[my] = total

        for t in range(1, N_DEV):
            @pl.when(my < t)
            def _(t=t):
                pltpu.make_async_remote_copy(
                    src_ref=totals_ref.at[my],
                    dst_ref=totals_ref.at[my],
                    send_sem=send_sems.at[t],
                    recv_sem=recv_sems.at[my],
                    device_id=(t,),
                    device_id_type=pl.DeviceIdType.MESH,
                ).start()

        acc = vals
        shift = 1
        while shift < m:
            pad = jnp.ones((shift, n), jnp.float32)
            acc = acc * jnp.concatenate([pad, acc[: m - shift, :]], axis=0)
            shift *= 2

        for src in range(N_DEV - 1):
            @pl.when(src < my)
            def _(src=src):
                pltpu.make_async_remote_copy(
                    src_ref=totals_ref.at[src],
                    dst_ref=totals_ref.at[src],
                    send_sem=send_sems.at[src],
                    recv_sem=recv_sems.at[src],
                    device_id=(0,),
                    device_id_type=pl.DeviceIdType.MESH,
                ).wait_recv()

        prefix = jnp.ones((1, n), jnp.float32)
        for src in range(N_DEV - 1):
            prefix = prefix * jnp.where(src < my, totals_ref[src], 1.0)

        out_ref[:, :] = acc * prefix

        for t in range(1, N_DEV):
            @pl.when(my < t)
            def _(t=t):
                pltpu.make_async_remote_copy(
                    src_ref=totals_ref.at[my],
                    dst_ref=totals_ref.at[my],
                    send_sem=send_sems.at[t],
                    recv_sem=recv_sems.at[my],
                    device_id=(t,),
                    device_id_type=pl.DeviceIdType.MESH,
                ).wait_send()

    return pl.pallas_call(
        body,
        out_shape=jax.ShapeDtypeStruct((m, n), jnp.float32),
        in_specs=[pl.BlockSpec(memory_space=pltpu.VMEM)],
        out_specs=pl.BlockSpec(memory_space=pltpu.VMEM),
        scratch_shapes=[
            pltpu.VMEM((N_DEV, 1, n), jnp.float32),
            pltpu.SemaphoreType.DMA((N_DEV,)),
            pltpu.SemaphoreType.DMA((N_DEV,)),
        ],
    )(x)


# device time: 4300 ns/iter; 3.1560x vs baseline; 2.5463x over previous
import jax
import jax.numpy as jnp
from jax import lax
from jax.experimental import pallas as pl
from jax.experimental.pallas import tpu as pltpu

N_DEV = 16


def kernel(x):
    m, n = x.shape

    def body(x_ref, out_ref, totals_ref, send_sems, recv_sems):
        my = lax.axis_index("i")

        barrier_sem = pltpu.get_barrier_semaphore()
        pl.semaphore_signal(
            barrier_sem,
            inc=1,
            device_id=(my,),
            device_id_type=pl.DeviceIdType.MESH,
        )
        pl.semaphore_wait(barrier_sem, 1)

        vals = x_ref[:, :]
        tot = vals
        rows = m
        while rows > 1:
            half = rows // 2
            tot = tot[:half, :] * tot[half:rows, :]
            rows = half
        total = tot[0:1, :]
        totals_ref[my] = total

        for t in range(1, N_DEV):
            @pl.when(my < t)
            def _(t=t):
                pltpu.make_async_remote_copy(
                    src_ref=totals_ref.at[my],
                    dst_ref=totals_ref.at[my],
                    send_sem=send_sems.at[t],
                    recv_sem=recv_sems.at[my],
                    device_id=(t,),
                    device_id_type=pl.DeviceIdType.MESH,
                ).start()

        acc = vals
        shift = 1
        while shift < m:
            pad = jnp.ones((shift, n), jnp.float32)
            acc = acc * jnp.concatenate([pad, acc[: m - shift, :]], axis=0)
            shift *= 2

        for src in range(N_DEV - 1):
            @pl.when(src < my)
            def _(src=src):
                pltpu.make_async_remote_copy(
                    src_ref=totals_ref.at[src],
                    dst_ref=totals_ref.at[src],
                    send_sem=send_sems.at[src],
                    recv_sem=recv_sems.at[src],
                    device_id=(0,),
                    device_id_type=pl.DeviceIdType.MESH,
                ).wait_recv()

        prefix = jnp.ones((1, n), jnp.float32)
        for src in range(N_DEV - 1):
            prefix = prefix * jnp.where(src < my, totals_ref[src], 1.0)

        out_ref[:, :] = acc * prefix

        for t in range(1, N_DEV):
            @pl.when(my < t)
            def _(t=t):
                pltpu.make_async_remote_copy(
                    src_ref=totals_ref.at[my],
                    dst_ref=totals_ref.at[my],
                    send_sem=send_sems.at[t],
                    recv_sem=recv_sems.at[my],
                    device_id=(t,),
                    device_id_type=pl.DeviceIdType.MESH,
                ).wait_send()

    return pl.pallas_call(
        body,
        out_shape=jax.ShapeDtypeStruct((m, n), jnp.float32),
        in_specs=[pl.BlockSpec(memory_space=pltpu.VMEM)],
        out_specs=pl.BlockSpec(memory_space=pltpu.VMEM),
        scratch_shapes=[
            pltpu.VMEM((N_DEV, 1, n), jnp.float32),
            pltpu.SemaphoreType.DMA((N_DEV,)),
            pltpu.SemaphoreType.DMA((N_DEV,)),
        ],
        compiler_params=pltpu.CompilerParams(collective_id=0),
    )(x)
